# baseline (speedup 1.0000x reference)
import jax
import jax.numpy as jnp
from jax import lax
from jax.experimental import pallas as pl
from jax.experimental.pallas import tpu as pltpu
from jax.experimental.pallas import tpu_sc as plsc

N = 10000
D = 128
A = 8
K = 8
M = A * N
NNZ = 1280000

NW = 32
ESL = NNZ // NW
EBLK = 2048
NBLK = 19
TAIL = ESL - NBLK * EBLK

NCH = 8
CH = 10240
LCAP = 416
SEGW = 16 * LCAP
NBATCH = SEGW // 128
TOT = NW * NCH * SEGW
TSIZ = TOT + NW * 128

NPASS = 4
TPR = CH // 16
ZR = 128
NZB = TPR // ZR

_mesh = plsc.VectorSubcoreMesh(core_axis_name="c", subcore_axis_name="s")


def _bucket_body(rows_hbm, cols_hbm, vals_hbm, seg_r, seg_c, seg_v,
                 e_r, e_c, e_v, sidx, sem):
    c = lax.axis_index("c")
    s = lax.axis_index("s")
    w = c * 16 + s
    lane = lax.iota(jnp.int32, 16)
    zero16 = jnp.zeros((16,), jnp.float32)

    def _fill_cv(g, _):
        e_c[pl.ds(g * 16, 16)] = lane + g * 16
        e_v[pl.ds(g * 16, 16)] = zero16
        return 0
    lax.fori_loop(0, EBLK // 16, _fill_cv, 0)
    for q in range(NCH):
        def _fill_r(g, _, _q=q):
            e_r[pl.ds(g * 16, 16)] = jnp.full((16,), _q * CH, jnp.int32)
            return 0
        lax.fori_loop(0, EBLK // 16, _fill_r, 0)
        rb = (w * NCH + q) * SEGW
        for t in range(SEGW // EBLK):
            pltpu.sync_copy(e_r, seg_r.at[pl.ds(rb + t * EBLK, EBLK)])
            pltpu.sync_copy(e_c, seg_c.at[pl.ds(rb + t * EBLK, EBLK)])
            pltpu.sync_copy(e_v, seg_v.at[pl.ds(rb + t * EBLK, EBLK)])
        rem = SEGW % EBLK
        fb = rb + (SEGW // EBLK) * EBLK
        pltpu.sync_copy(e_r.at[pl.ds(0, rem)], seg_r.at[pl.ds(fb, rem)])
        pltpu.sync_copy(e_c.at[pl.ds(0, rem)], seg_c.at[pl.ds(fb, rem)])
        pltpu.sync_copy(e_v.at[pl.ds(0, rem)], seg_v.at[pl.ds(fb, rem)])

    bq = [(w * NCH + q) * SEGW + lane * LCAP for q in range(NCH)]

    def _group(g, cnts):
        r = e_r[pl.ds(g * 16, 16)]
        x = r >> 11
        qid = (x * 13108) >> 16
        sv = jnp.zeros((16,), jnp.int32)
        new = []
        for q in range(NCH):
            sel = qid == q
            cq = cnts[q]
            sv = sv + jnp.where(sel, bq[q] + jnp.minimum(cq, LCAP - 1), 0)
            new.append(cq + jnp.where(sel, 1, 0))
        sidx[g // 8, pl.ds((g % 8) * 16, 16)] = sv
        return tuple(new)

    def _scatter(nrows):
        cps = []
        for jr in range(nrows):
            so = jr * 128
            cps.append(pltpu.async_copy(e_r.at[pl.ds(so, 128)],
                                        seg_r.at[sidx.at[jr]], sem))
            cps.append(pltpu.async_copy(e_c.at[pl.ds(so, 128)],
                                        seg_c.at[sidx.at[jr]], sem))
            cps.append(pltpu.async_copy(e_v.at[pl.ds(so, 128)],
                                        seg_v.at[sidx.at[jr]], sem))
        for cp in cps:
            cp.wait()

    cnts0 = tuple(jnp.zeros((16,), jnp.int32) for _ in range(NCH))

    def _blk(b, cnts):
        eoff = w * ESL + b * EBLK
        pltpu.sync_copy(rows_hbm.at[pl.ds(eoff, EBLK)], e_r)
        pltpu.sync_copy(cols_hbm.at[pl.ds(eoff, EBLK)], e_c)
        pltpu.sync_copy(vals_hbm.at[pl.ds(eoff, EBLK)], e_v)
        cnts = lax.fori_loop(0, EBLK // 16, _group, cnts)
        _scatter(16)
        return cnts
    cnts = lax.fori_loop(0, NBLK, _blk, cnts0)

    eoff = w * ESL + NBLK * EBLK
    pltpu.sync_copy(rows_hbm.at[pl.ds(eoff, TAIL)], e_r.at[pl.ds(0, TAIL)])
    pltpu.sync_copy(cols_hbm.at[pl.ds(eoff, TAIL)], e_c.at[pl.ds(0, TAIL)])
    pltpu.sync_copy(vals_hbm.at[pl.ds(eoff, TAIL)], e_v.at[pl.ds(0, TAIL)])
    lax.fori_loop(0, TAIL // 16, _group, cnts)
    for t in range(4):
        sidx[8, pl.ds(64 + t * 16, 16)] = TOT + w * 128 + lane + t * 16
    _scatter(9)


_bucket = pl.kernel(
    _bucket_body,
    out_type=(jax.ShapeDtypeStruct((TSIZ,), jnp.int32),
              jax.ShapeDtypeStruct((TSIZ,), jnp.int32),
              jax.ShapeDtypeStruct((TSIZ,), jnp.float32)),
    mesh=_mesh,
    scratch_types=[
        pltpu.VMEM((EBLK,), jnp.int32),
        pltpu.VMEM((EBLK,), jnp.int32),
        pltpu.VMEM((EBLK,), jnp.float32),
        pltpu.VMEM((16, 128), jnp.int32),
        pltpu.SemaphoreType.DMA,
    ],
    name="acs_bucket",
)


NB2 = 2 * NBATCH


def _spmm_body(first, seg_r, seg_c, seg_v, tprev, tprev2, out,
               idxr0, idxr1, sidx0, sidx1, cbuf0, cbuf1,
               vxp, vxbuf0, vxbuf1, gbuf0, gbuf1, acc, *sems):
    idxrs = (idxr0, idxr1)
    sidxs = (sidx0, sidx1)
    cbufs = (cbuf0, cbuf1)
    vxbufs = (vxbuf0, vxbuf1)
    gbufs = (gbuf0, gbuf1)
    semSc = sems[0:2]
    semG = sems[2:4]
    semI = sems[4:6]
    semC = sems[6:8]
    semV = sems[8:10]
    gbufA, gbufB = gbuf0, gbuf1
    c = lax.axis_index("c")
    s = lax.axis_index("s")
    for t2 in range(16):
        for hb in range(8):
            vxp[t2, pl.ds(hb * 16, 16)] = jnp.full((16,), t2 * 8 + hb,
                                                   jnp.int32)

    def _drain(src, dst, sem):
        pltpu.make_async_copy(src, dst, sem).wait()

    for p in range(NPASS):
        q = 2 * p + c
        base = q * CH

        def _zrow(i, _):
            for h in range(8):
                gbufB[i, pl.ds(h * 16, 16)] = jnp.zeros((16,), jnp.float32)
            return 0
        lax.fori_loop(0, ZR, _zrow, 0)

        def _z(z, _):
            pltpu.sync_copy(gbufB, acc.at[pl.ds(s * TPR + z * ZR, ZR)])
            return 0
        lax.fori_loop(0, NZB, _z, 0)
        plsc.subcore_barrier()

        def _boff(jb):
            hi = jb >= NBATCH
            wo = jnp.where(hi, 16, 0)
            col = jb - jnp.where(hi, NBATCH, 0)
            return ((s + wo) * NCH + q) * SEGW + col * 128

        def _stage(jb, i):
            boff = _boff(jb)
            pltpu.async_copy(seg_r.at[pl.ds(boff, 128)], idxrs[i].at[0],
                             semI[i])
            pltpu.async_copy(seg_c.at[pl.ds(boff, 128)], cbufs[i].at[0],
                             semC[i])
            for t2 in range(16):
                pltpu.async_copy(seg_v.at[pl.ds(boff, 128)].at[vxp.at[t2]],
                                 vxbufs[i].at[pl.ds(t2 * 128, 128)], semV[i])

        def _batch(jb, i):
            i1 = 1 - i
            gb = gbufs[i]
            _drain(seg_r.at[pl.ds(0, 128)], idxrs[i].at[0], semI[i])
            for gp in range(8):
                sidxs[i][0, pl.ds(gp * 16, 16)] = (
                    idxrs[i][0, pl.ds(gp * 16, 16)] - base)
            _drain(seg_v.at[pl.ds(0, 2048)], vxbufs[i], semV[i])

            @pl.when(jb + 1 < NB2)
            def _():
                @pl.when(jb >= 1)
                def _():
                    _drain(tprev.at[pl.ds(0, 128)], gbufs[i1], semSc[i1])
                _drain(seg_c.at[pl.ds(0, 128)], cbufs[i1].at[0], semC[i1])
                pltpu.async_copy(tprev.at[cbufs[i1].at[0]], gbufs[i1],
                                 semG[i1])

            _drain(tprev.at[pl.ds(0, 128)], gb, semG[i])

            def _srow(j4, _):
                for u in range(4):
                    j = j4 * 4 + u
                    vs = vxbufs[i][pl.ds(16 * j, 16)]
                    if not first:
                        vs = vs + vs
                    for h in range(8):
                        gb[j, pl.ds(16 * h, 16)] = (
                            gb[j, pl.ds(16 * h, 16)] * vs)
                return 0
            lax.fori_loop(0, 32, _srow, 0)

            @pl.when(jb + 2 < NB2)
            def _():
                _stage(jb + 2, i)

            pltpu.async_copy(gb, acc.at[sidxs[i].at[0]], semSc[i], add=True)

        _stage(jnp.int32(0), 0)
        _stage(jnp.int32(1), 1)
        _drain(seg_c.at[pl.ds(0, 128)], cbufs[0].at[0], semC[0])
        pltpu.async_copy(tprev.at[cbufs[0].at[0]], gbufs[0], semG[0])

        def _pair(m, _):
            _batch(2 * m, 0)
            _batch(2 * m + 1, 1)
            return 0
        lax.fori_loop(0, NB2 // 2, _pair, 0)
        for i in range(2):
            _drain(tprev.at[pl.ds(0, 128)], gbufs[i], semSc[i])
        plsc.subcore_barrier()

        def _epi(z, _):
            arow = s * TPR + z * ZR
            grow = base + arow

            @pl.when(grow < M)
            def _():
                if first:
                    pltpu.sync_copy(acc.at[pl.ds(arow, ZR)],
                                    out.at[pl.ds(grow, ZR)])
                else:
                    pltpu.sync_copy(acc.at[pl.ds(arow, ZR)], gbufA)
                    pltpu.sync_copy(tprev2.at[pl.ds(grow, ZR)], gbufB)

                    def _crow(i, _):
                        for h in range(8):
                            gbufA[i, pl.ds(h * 16, 16)] = (
                                gbufA[i, pl.ds(h * 16, 16)]
                                - gbufB[i, pl.ds(h * 16, 16)])
                        return 0
                    lax.fori_loop(0, ZR, _crow, 0)
                    pltpu.sync_copy(gbufA, out.at[pl.ds(grow, ZR)])
            return 0
        lax.fori_loop(0, NZB, _epi, 0)


def _make_spmm(first):
    scratch = (
        [pltpu.VMEM((1, 128), jnp.int32)] * 2
        + [pltpu.VMEM((1, 128), jnp.int32)] * 2
        + [pltpu.VMEM((1, 128), jnp.int32)] * 2
        + [pltpu.VMEM((16, 128), jnp.int32)]
        + [pltpu.VMEM((2048,), jnp.float32)] * 2
        + [pltpu.VMEM((128, D), jnp.float32)] * 2
        + [pltpu.VMEM_SHARED((CH, D), jnp.float32)]
        + [pltpu.SemaphoreType.DMA] * 10)
    if first:
        def body(seg_r, seg_c, seg_v, tprev, out, *sc):
            _spmm_body(True, seg_r, seg_c, seg_v, tprev, None, out, *sc)
    else:
        def body(seg_r, seg_c, seg_v, tprev, tprev2, out, *sc):
            _spmm_body(False, seg_r, seg_c, seg_v, tprev, tprev2, out, *sc)
    return pl.kernel(
        body,
        out_type=jax.ShapeDtypeStruct((M, D), jnp.float32),
        mesh=_mesh,
        scratch_types=scratch,
        name="acs_spmm1" if first else "acs_spmm",
    )


_spmm_first = _make_spmm(True)
_spmm_next = _make_spmm(False)


BN = 1000


def _mm_body(x_ref, t1, t2, t3, t4, t5, t6, t7, w_ref, b_ref, o_ref):
    a = pl.program_id(1)
    acc = jnp.dot(x_ref[...], w_ref[0], preferred_element_type=jnp.float32)
    for k, t in enumerate((t1, t2, t3, t4, t5, t6, t7), start=1):
        acc += jnp.dot(t[...], w_ref[k], preferred_element_type=jnp.float32)

    @pl.when(a == 0)
    def _():
        o_ref[...] = acc + b_ref[...]

    @pl.when(a != 0)
    def _():
        o_ref[...] = o_ref[...] + acc


def _matmul(x, txs, W, b2d):
    nblk = N // BN
    tx_spec = pl.BlockSpec((BN, D), lambda n, a: (a * (N // BN) + n, 0))
    return pl.pallas_call(
        _mm_body,
        grid=(nblk, A),
        in_specs=[pl.BlockSpec((BN, D), lambda n, a: (n, 0))]
        + [tx_spec] * 7
        + [pl.BlockSpec((K, D, D), lambda n, a: (0, a, 0)),
           pl.BlockSpec((1, D), lambda n, a: (0, 0))],
        out_specs=pl.BlockSpec((BN, D), lambda n, a: (n, 0)),
        out_shape=jax.ShapeDtypeStruct((N, D), jnp.float32),
        compiler_params=pltpu.CompilerParams(
            dimension_semantics=("parallel", "arbitrary")),
    )(x, *txs, W, b2d)


def kernel(x, Ls_indices, Ls_values, W, b):
    rows = Ls_indices[0].astype(jnp.int32)
    cols = Ls_indices[1].astype(jnp.int32)
    vals = Ls_values
    seg_r, seg_c, seg_v = _bucket(rows, cols, vals)
    tx0 = jnp.tile(x, (A, 1))
    txs = [tx0, _spmm_first(seg_r, seg_c, seg_v, tx0)]
    for k in range(2, K):
        txs.append(_spmm_next(seg_r, seg_c, seg_v, txs[k - 1], txs[k - 2]))
    return _matmul(x, txs[1:], W, b.reshape(1, D))


# speedup vs baseline: 1.1393x; 1.1393x over previous
import jax
import jax.numpy as jnp
from jax import lax
from jax.experimental import pallas as pl
from jax.experimental.pallas import tpu as pltpu
from jax.experimental.pallas import tpu_sc as plsc

N = 10000
D = 128
A = 8
K = 8
M = A * N
NNZ = 1280000

NW = 32
ESL = NNZ // NW
EBLK = 2048
NBLK = 19
TAIL = ESL - NBLK * EBLK

NCH = 8
CH = 10240
LCAP = 416
SEGW = 16 * LCAP
NBATCH = SEGW // 128
TOT = NW * NCH * SEGW
TSIZ = TOT + NW * 128

NPASS = 4
TPR = CH // 16
ZR = 128
NZB = TPR // ZR

_mesh = plsc.VectorSubcoreMesh(core_axis_name="c", subcore_axis_name="s")


def _bucket_body(rows_hbm, cols_hbm, vals_hbm, seg_r, seg_c, seg_v,
                 e_r, e_c, e_v, sidx, sem):
    c = lax.axis_index("c")
    s = lax.axis_index("s")
    w = c * 16 + s
    lane = lax.iota(jnp.int32, 16)
    zero16 = jnp.zeros((16,), jnp.float32)

    def _fill_cv(g, _):
        e_c[pl.ds(g * 16, 16)] = lane + g * 16
        e_v[pl.ds(g * 16, 16)] = zero16
        return 0
    lax.fori_loop(0, EBLK // 16, _fill_cv, 0)
    for q in range(NCH):
        def _fill_r(g, _, _q=q):
            e_r[pl.ds(g * 16, 16)] = jnp.full((16,), _q * CH, jnp.int32)
            return 0
        lax.fori_loop(0, EBLK // 16, _fill_r, 0)
        rb = (w * NCH + q) * SEGW
        for t in range(SEGW // EBLK):
            pltpu.sync_copy(e_r, seg_r.at[pl.ds(rb + t * EBLK, EBLK)])
            pltpu.sync_copy(e_c, seg_c.at[pl.ds(rb + t * EBLK, EBLK)])
            pltpu.sync_copy(e_v, seg_v.at[pl.ds(rb + t * EBLK, EBLK)])
        rem = SEGW % EBLK
        fb = rb + (SEGW // EBLK) * EBLK
        pltpu.sync_copy(e_r.at[pl.ds(0, rem)], seg_r.at[pl.ds(fb, rem)])
        pltpu.sync_copy(e_c.at[pl.ds(0, rem)], seg_c.at[pl.ds(fb, rem)])
        pltpu.sync_copy(e_v.at[pl.ds(0, rem)], seg_v.at[pl.ds(fb, rem)])

    bq = [(w * NCH + q) * SEGW + lane * LCAP for q in range(NCH)]

    def _group(g, cnts):
        r = e_r[pl.ds(g * 16, 16)]
        x = r >> 11
        qid = (x * 13108) >> 16
        sv = jnp.zeros((16,), jnp.int32)
        new = []
        for q in range(NCH):
            sel = qid == q
            cq = cnts[q]
            sv = sv + jnp.where(sel, bq[q] + jnp.minimum(cq, LCAP - 1), 0)
            new.append(cq + jnp.where(sel, 1, 0))
        sidx[g // 8, pl.ds((g % 8) * 16, 16)] = sv
        return tuple(new)

    def _scatter(nrows):
        cps = []
        for jr in range(nrows):
            so = jr * 128
            cps.append(pltpu.async_copy(e_r.at[pl.ds(so, 128)],
                                        seg_r.at[sidx.at[jr]], sem))
            cps.append(pltpu.async_copy(e_c.at[pl.ds(so, 128)],
                                        seg_c.at[sidx.at[jr]], sem))
            cps.append(pltpu.async_copy(e_v.at[pl.ds(so, 128)],
                                        seg_v.at[sidx.at[jr]], sem))
        for cp in cps:
            cp.wait()

    cnts0 = tuple(jnp.zeros((16,), jnp.int32) for _ in range(NCH))

    def _blk(b, cnts):
        eoff = w * ESL + b * EBLK
        pltpu.sync_copy(rows_hbm.at[pl.ds(eoff, EBLK)], e_r)
        pltpu.sync_copy(cols_hbm.at[pl.ds(eoff, EBLK)], e_c)
        pltpu.sync_copy(vals_hbm.at[pl.ds(eoff, EBLK)], e_v)
        cnts = lax.fori_loop(0, EBLK // 16, _group, cnts)
        _scatter(16)
        return cnts
    cnts = lax.fori_loop(0, NBLK, _blk, cnts0)

    eoff = w * ESL + NBLK * EBLK
    pltpu.sync_copy(rows_hbm.at[pl.ds(eoff, TAIL)], e_r.at[pl.ds(0, TAIL)])
    pltpu.sync_copy(cols_hbm.at[pl.ds(eoff, TAIL)], e_c.at[pl.ds(0, TAIL)])
    pltpu.sync_copy(vals_hbm.at[pl.ds(eoff, TAIL)], e_v.at[pl.ds(0, TAIL)])
    lax.fori_loop(0, TAIL // 16, _group, cnts)
    for t in range(4):
        sidx[8, pl.ds(64 + t * 16, 16)] = TOT + w * 128 + lane + t * 16
    _scatter(9)


_bucket = pl.kernel(
    _bucket_body,
    out_type=(jax.ShapeDtypeStruct((TSIZ,), jnp.int32),
              jax.ShapeDtypeStruct((TSIZ,), jnp.int32),
              jax.ShapeDtypeStruct((TSIZ,), jnp.float32)),
    mesh=_mesh,
    scratch_types=[
        pltpu.VMEM((EBLK,), jnp.int32),
        pltpu.VMEM((EBLK,), jnp.int32),
        pltpu.VMEM((EBLK,), jnp.float32),
        pltpu.VMEM((16, 128), jnp.int32),
        pltpu.SemaphoreType.DMA,
    ],
    name="acs_bucket",
)


NB2 = 2 * NBATCH


def _spmm_body(first, seg_r, seg_c, seg_v, tprev, tprev2, out,
               idxrA, idxrB, sidxA, sidxB, cbufA, cbufB, vxp,
               vxbufA, vxbufB, gbufA, gbufB, acc,
               semScA, semScB, semGA, semGB, semIA, semIB,
               semCA, semCB, semVA, semVB):
    c = lax.axis_index("c")
    s = lax.axis_index("s")
    for t2 in range(16):
        for hb in range(8):
            vxp[t2, pl.ds(hb * 16, 16)] = jnp.full((16,), t2 * 8 + hb,
                                                   jnp.int32)

    def _drain(src, dst, sem):
        pltpu.make_async_copy(src, dst, sem).wait()

    for p in range(NPASS):
        q = 2 * p + c
        base = q * CH

        def _zrow(i, _):
            for h in range(8):
                gbufB[i, pl.ds(h * 16, 16)] = jnp.zeros((16,), jnp.float32)
            return 0
        lax.fori_loop(0, ZR, _zrow, 0)

        def _z(z, _):
            pltpu.sync_copy(gbufB, acc.at[pl.ds(s * TPR + z * ZR, ZR)])
            return 0
        lax.fori_loop(0, NZB, _z, 0)
        plsc.subcore_barrier()

        def _boff(jb):
            hi = jb >= NBATCH
            wo = jnp.where(hi, 16, 0)
            col = jb - jnp.where(hi, NBATCH, 0)
            return ((s + wo) * NCH + q) * SEGW + col * 128

        def _stage(jb, idxr, cbuf, vxbuf, semI, semC, semV):
            boff = _boff(jb)
            pltpu.async_copy(seg_r.at[pl.ds(boff, 128)], idxr.at[0], semI)
            pltpu.async_copy(seg_c.at[pl.ds(boff, 128)], cbuf.at[0], semC)
            for t2 in range(16):
                pltpu.async_copy(seg_v.at[pl.ds(boff, 128)].at[vxp.at[t2]],
                                 vxbuf.at[pl.ds(t2 * 128, 128)], semV)

        def _batch(jb, idxr, sidx2, cbuf, vxbuf, gb, semSc, semG, semI,
                   semC, semV, o_cbuf, o_gbuf, o_semSc, o_semG, o_semC):
            _drain(seg_r.at[pl.ds(0, 128)], idxr.at[0], semI)
            for gp in range(8):
                sidx2[0, pl.ds(gp * 16, 16)] = (
                    idxr[0, pl.ds(gp * 16, 16)] - base)
            _drain(seg_v.at[pl.ds(0, 2048)], vxbuf, semV)
            _drain(tprev.at[pl.ds(0, 128)], gb, semG)

            def _srow(j4, _):
                for u in range(4):
                    j = j4 * 4 + u
                    vs = vxbuf[pl.ds(16 * j, 16)]
                    if not first:
                        vs = vs + vs
                    for h in range(8):
                        gb[j, pl.ds(16 * h, 16)] = (
                            gb[j, pl.ds(16 * h, 16)] * vs)
                return 0
            lax.fori_loop(0, 32, _srow, 0)

            @pl.when(jb + 2 < NB2)
            def _():
                _stage(jb + 2, idxr, cbuf, vxbuf, semI, semC, semV)

            @pl.when(jb + 1 < NB2)
            def _():
                @pl.when(jb >= 1)
                def _():
                    _drain(tprev.at[pl.ds(0, 128)], o_gbuf, o_semSc)
                _drain(seg_c.at[pl.ds(0, 128)], o_cbuf.at[0], o_semC)
                pltpu.async_copy(tprev.at[o_cbuf.at[0]], o_gbuf, o_semG)

            pltpu.async_copy(gb, acc.at[sidx2.at[0]], semSc, add=True)

        _stage(jnp.int32(0), idxrA, cbufA, vxbufA, semIA, semCA, semVA)
        _stage(jnp.int32(1), idxrB, cbufB, vxbufB, semIB, semCB, semVB)
        _drain(seg_c.at[pl.ds(0, 128)], cbufA.at[0], semCA)
        pltpu.async_copy(tprev.at[cbufA.at[0]], gbufA, semGA)

        def _pair(m, _):
            _batch(2 * m, idxrA, sidxA, cbufA, vxbufA, gbufA,
                   semScA, semGA, semIA, semCA, semVA,
                   cbufB, gbufB, semScB, semGB, semCB)
            _batch(2 * m + 1, idxrB, sidxB, cbufB, vxbufB, gbufB,
                   semScB, semGB, semIB, semCB, semVB,
                   cbufA, gbufA, semScA, semGA, semCA)
            return 0
        lax.fori_loop(0, NB2 // 2, _pair, 0)
        _drain(tprev.at[pl.ds(0, 128)], gbufA, semScA)
        _drain(tprev.at[pl.ds(0, 128)], gbufB, semScB)
        plsc.subcore_barrier()

        def _epi(z, _):
            arow = s * TPR + z * ZR
            grow = base + arow

            @pl.when(grow < M)
            def _():
                if first:
                    pltpu.sync_copy(acc.at[pl.ds(arow, ZR)],
                                    out.at[pl.ds(grow, ZR)])
                else:
                    pltpu.sync_copy(acc.at[pl.ds(arow, ZR)], gbufA)
                    pltpu.sync_copy(tprev2.at[pl.ds(grow, ZR)], gbufB)

                    def _crow(i, _):
                        for h in range(8):
                            gbufA[i, pl.ds(h * 16, 16)] = (
                                gbufA[i, pl.ds(h * 16, 16)]
                                - gbufB[i, pl.ds(h * 16, 16)])
                        return 0
                    lax.fori_loop(0, ZR, _crow, 0)
                    pltpu.sync_copy(gbufA, out.at[pl.ds(grow, ZR)])
            return 0
        lax.fori_loop(0, NZB, _epi, 0)


def _make_spmm(first):
    scratch = [
        pltpu.VMEM((1, 128), jnp.int32),
        pltpu.VMEM((1, 128), jnp.int32),
        pltpu.VMEM((1, 128), jnp.int32),
        pltpu.VMEM((1, 128), jnp.int32),
        pltpu.VMEM((1, 128), jnp.int32),
        pltpu.VMEM((1, 128), jnp.int32),
        pltpu.VMEM((16, 128), jnp.int32),
        pltpu.VMEM((2048,), jnp.float32),
        pltpu.VMEM((2048,), jnp.float32),
        pltpu.VMEM((128, D), jnp.float32),
        pltpu.VMEM((128, D), jnp.float32),
        pltpu.VMEM_SHARED((CH, D), jnp.float32),
    ] + [pltpu.SemaphoreType.DMA] * 10
    if first:
        def body(seg_r, seg_c, seg_v, tprev, out, *sc):
            _spmm_body(True, seg_r, seg_c, seg_v, tprev, None, out, *sc)
    else:
        def body(seg_r, seg_c, seg_v, tprev, tprev2, out, *sc):
            _spmm_body(False, seg_r, seg_c, seg_v, tprev, tprev2, out, *sc)
    return pl.kernel(
        body,
        out_type=jax.ShapeDtypeStruct((M, D), jnp.float32),
        mesh=_mesh,
        scratch_types=scratch,
        name="acs_spmm1" if first else "acs_spmm",
    )


_spmm_first = _make_spmm(True)
_spmm_next = _make_spmm(False)


BN = 1000


def _mm_body(x_ref, t1, t2, t3, t4, t5, t6, t7, w_ref, b_ref, o_ref):
    a = pl.program_id(1)
    acc = jnp.dot(x_ref[...], w_ref[0], preferred_element_type=jnp.float32)
    for k, t in enumerate((t1, t2, t3, t4, t5, t6, t7), start=1):
        acc += jnp.dot(t[...], w_ref[k], preferred_element_type=jnp.float32)

    @pl.when(a == 0)
    def _():
        o_ref[...] = acc + b_ref[...]

    @pl.when(a != 0)
    def _():
        o_ref[...] = o_ref[...] + acc


def _matmul(x, txs, W, b2d):
    nblk = N // BN
    tx_spec = pl.BlockSpec((BN, D), lambda n, a: (a * (N // BN) + n, 0))
    return pl.pallas_call(
        _mm_body,
        grid=(nblk, A),
        in_specs=[pl.BlockSpec((BN, D), lambda n, a: (n, 0))]
        + [tx_spec] * 7
        + [pl.BlockSpec((K, D, D), lambda n, a: (0, a, 0)),
           pl.BlockSpec((1, D), lambda n, a: (0, 0))],
        out_specs=pl.BlockSpec((BN, D), lambda n, a: (n, 0)),
        out_shape=jax.ShapeDtypeStruct((N, D), jnp.float32),
        compiler_params=pltpu.CompilerParams(
            dimension_semantics=("parallel", "arbitrary")),
    )(x, *txs, W, b2d)


def kernel(x, Ls_indices, Ls_values, W, b):
    rows = Ls_indices[0].astype(jnp.int32)
    cols = Ls_indices[1].astype(jnp.int32)
    vals = Ls_values
    seg_r, seg_c, seg_v = _bucket(rows, cols, vals)
    tx0 = jnp.tile(x, (A, 1))
    txs = [tx0, _spmm_first(seg_r, seg_c, seg_v, tx0)]
    for k in range(2, K):
        txs.append(_spmm_next(seg_r, seg_c, seg_v, txs[k - 1], txs[k - 2]))
    return _matmul(x, txs[1:], W, b.reshape(1, D))
